# Initial kernel scaffold; baseline (speedup 1.0000x reference)
import jax
import jax.numpy as jnp
from jax.experimental import pallas as pl


def kernel(pos, viewdirs, tables, W1, W2, C1, C2, C3):
    raise NotImplementedError("write your pallas kernel here")


# speedup vs baseline: 1.2008x; 1.2008x over previous
import functools

import jax
import jax.numpy as jnp
import numpy as np
from jax import lax
from jax.experimental import pallas as pl
from jax.experimental.pallas import tpu as pltpu
from jax.experimental.pallas import tpu_sc as plsc

N_PTS = 262144
NL = 16
FPL = 2
T = 1 << 19
MASK = T - 1
PRIME1 = np.int32(np.uint32(2654435761).astype(np.int64) - (1 << 32))
PRIME2 = np.int32(805459861)

NC, NS, LANES = 2, 16, 16
NW = NC * NS
PTS_PER_W = N_PTS // NW
P = 128
NCH = PTS_PER_W // P
NV = P // LANES
NR = 8 * P

SCALE0 = np.float32(15.0)


def _encode_body(tf0, tf1, px, py, pz, out,
                 pxv, pyv, pzv, idxb, rows0, rows1, outb, sem0, sem1):
    wid = lax.axis_index("s") * NC + lax.axis_index("c")
    base = wid * PTS_PER_W
    for src, dst in ((px, pxv), (py, pyv), (pz, pzv)):
        pltpu.sync_copy(src.at[pl.ds(base, PTS_PER_W)], dst)

    iota = lax.iota(jnp.int32, LANES)

    @pl.loop(0, NCH)
    def _chunk(ch):
        pbase = ch * P

        @pl.loop(0, NL, init_carry=jnp.full((LANES,), SCALE0, jnp.float32))
        def _level(l, scale):
            l_vec = jnp.full((LANES,), 0, jnp.int32) + l
            is_dense = l_vec < 4
            res = jnp.where(l_vec < 1, 16,
                            jnp.where(l_vec < 2, 24,
                                      jnp.where(l_vec < 3, 36, 55)))
            res1 = res - 1
            res2 = res * res
            lT = l_vec * T

            for v in range(NV):
                sl = pl.ds(pbase + v * LANES, LANES)
                fx = pxv[sl] * scale + 0.5
                fy = pyv[sl] * scale + 0.5
                fz = pzv[sl] * scale + 0.5
                xi = fx.astype(jnp.int32)
                yi = fy.astype(jnp.int32)
                zi = fz.astype(jnp.int32)
                gx = (xi, xi + 1)
                gy0 = yi * PRIME1
                gy = (gy0, gy0 + PRIME1)
                gz0 = zi * PRIME2
                gz = (gz0, gz0 + PRIME2)
                dx = (jnp.minimum(xi, res1), jnp.minimum(xi + 1, res1))
                dy0 = jnp.minimum(yi, res1) * res
                dy = (dy0, jnp.minimum(yi + 1, res1) * res)
                dz0 = jnp.minimum(zi, res1) * res2
                dz = (dz0, jnp.minimum(zi + 1, res1) * res2)
                for c in range(8):
                    bx, by, bz = c & 1, (c >> 1) & 1, (c >> 2) & 1
                    h = (gx[bx] ^ gy[by] ^ gz[bz]) & MASK
                    dn = dx[bx] + dy[by] + dz[bz]
                    idx = jnp.where(is_dense, dn, h) + lT
                    idxb[pl.ds(c * P + v * LANES, LANES)] = idx

            cp0 = pltpu.async_copy(tf0.at[idxb], rows0, sem0)
            cp1 = pltpu.async_copy(tf1.at[idxb], rows1, sem1)
            cp0.wait()
            cp1.wait()

            col0 = jnp.full((LANES,), 0, jnp.int32) + 2 * l
            for v in range(NV):
                sl = pl.ds(pbase + v * LANES, LANES)
                fx = pxv[sl] * scale + 0.5
                fy = pyv[sl] * scale + 0.5
                fz = pzv[sl] * scale + 0.5
                wx1 = fx - fx.astype(jnp.int32).astype(jnp.float32)
                wy1 = fy - fy.astype(jnp.int32).astype(jnp.float32)
                wz1 = fz - fz.astype(jnp.int32).astype(jnp.float32)
                wx = (1.0 - wx1, wx1)
                wy = (1.0 - wy1, wy1)
                wz = (1.0 - wz1, wz1)
                pyz = tuple(wy[j & 1] * wz[j >> 1] for j in range(4))
                acc0 = jnp.zeros((LANES,), jnp.float32)
                acc1 = jnp.zeros((LANES,), jnp.float32)
                for c in range(8):
                    wc = wx[c & 1] * pyz[c >> 1]
                    d0 = iota + (c * P + v * LANES)
                    f0 = plsc.load_gather(rows0, [d0])
                    f1 = plsc.load_gather(rows1, [d0])
                    acc0 = acc0 + wc * f0
                    acc1 = acc1 + wc * f1
                rowi = iota * (2 * NL) + (v * LANES * 2 * NL) + col0
                plsc.store_scatter(outb, [rowi], acc0)
                plsc.store_scatter(outb, [rowi + 1], acc1)

            return scale * 1.5 + 0.5

        pltpu.sync_copy(outb, out.at[pl.ds((base + pbase) * 2 * NL, P * 2 * NL)])


def _encode(tf0, tf1, px, py, pz):
    mesh = plsc.VectorSubcoreMesh(core_axis_name="c", subcore_axis_name="s")
    return pl.kernel(
        _encode_body,
        out_type=jax.ShapeDtypeStruct((N_PTS * 2 * NL,), jnp.float32),
        mesh=mesh,
        compiler_params=pltpu.CompilerParams(
            use_tc_tiling_on_sc=False, needs_layout_passes=False),
        scratch_types=[
            pltpu.VMEM((PTS_PER_W,), jnp.float32),
            pltpu.VMEM((PTS_PER_W,), jnp.float32),
            pltpu.VMEM((PTS_PER_W,), jnp.float32),
            pltpu.VMEM((NR,), jnp.int32),
            pltpu.VMEM((NR,), jnp.float32),
            pltpu.VMEM((NR,), jnp.float32),
            pltpu.VMEM((P * 2 * NL,), jnp.float32),
            pltpu.SemaphoreType.DMA,
            pltpu.SemaphoreType.DMA,
        ],
    )(tf0, tf1, px, py, pz)


def _sh16(d):
    x, y, z = d[:, 0:1], d[:, 1:2], d[:, 2:3]
    xy, xz, yz = x * y, x * z, y * z
    x2, y2, z2 = x * x, y * y, z * z
    return jnp.concatenate([
        jnp.full_like(x, 0.28209479177387814),
        -0.48860251190291987 * y,
        0.48860251190291987 * z,
        -0.48860251190291987 * x,
        1.0925484305920792 * xy,
        -1.0925484305920792 * yz,
        0.94617469575755997 * z2 - 0.31539156525251999,
        -1.0925484305920792 * xz,
        0.54627421529603959 * (x2 - y2),
        0.59004358992664352 * y * (-3.0 * x2 + y2),
        2.8906114426405538 * xy * z,
        0.45704579946446572 * y * (1.0 - 5.0 * z2),
        0.3731763325901154 * z * (5.0 * z2 - 3.0),
        0.45704579946446572 * x * (1.0 - 5.0 * z2),
        1.4453057213202769 * z * (x2 - y2),
        0.59004358992664352 * x * (-x2 + 3.0 * y2),
    ], axis=1)


def _mlp_body(enc, vd, W1, W2, C1, C2, C3, rgb, sigma):
    f32 = jnp.float32
    h = jnp.maximum(jnp.dot(enc[...], W1[...], preferred_element_type=f32), 0.0)
    feat = jnp.maximum(jnp.dot(h, W2[...], preferred_element_type=f32), 0.0)
    sigma[...] = feat[:, 0:1]
    sh = _sh16(vd[...] * 2.0 - 1.0)
    xcat = jnp.concatenate([feat, sh], axis=1)
    h2 = jnp.maximum(jnp.dot(xcat, C1[...], preferred_element_type=f32), 0.0)
    h3 = jnp.maximum(jnp.dot(h2, C2[...], preferred_element_type=f32), 0.0)
    rgb[...] = jax.nn.sigmoid(jnp.dot(h3, C3[...], preferred_element_type=f32))


def _mlp(enc, vd, W1, W2, C1, C2, C3, blk=4096):
    nblk = N_PTS // blk
    wspec = lambda shape: pl.BlockSpec(shape, lambda i: (0, 0))
    return pl.pallas_call(
        _mlp_body,
        grid=(nblk,),
        in_specs=[
            pl.BlockSpec((blk, 32), lambda i: (i, 0)),
            pl.BlockSpec((blk, 3), lambda i: (i, 0)),
            wspec((32, 64)), wspec((64, 16)),
            wspec((32, 64)), wspec((64, 64)), wspec((64, 3)),
        ],
        out_specs=[
            pl.BlockSpec((blk, 3), lambda i: (i, 0)),
            pl.BlockSpec((blk, 1), lambda i: (i, 0)),
        ],
        out_shape=[
            jax.ShapeDtypeStruct((N_PTS, 3), jnp.float32),
            jax.ShapeDtypeStruct((N_PTS, 1), jnp.float32),
        ],
    )(enc, vd, W1, W2, C1, C2, C3)


def kernel(pos, viewdirs, tables, W1, W2, C1, C2, C3):
    tf0 = tables[:, :, 0].reshape(NL * T)
    tf1 = tables[:, :, 1].reshape(NL * T)
    enc = _encode(tf0, tf1, pos[:, 0], pos[:, 1], pos[:, 2])
    enc = enc.reshape(N_PTS, 2 * NL)
    rgb, sigma = _mlp(enc, viewdirs, W1, W2, C1, C2, C3)
    return (rgb, sigma)
